# baseline (speedup 1.0000x reference)
import jax
import jax.numpy as jnp
from jax import lax
from jax.experimental import pallas as pl
from jax.experimental.pallas import tpu as pltpu
from jax.experimental.pallas import tpu_sc as plsc

NC = 2
NS = 16
NW = NC * NS
DP = 128
D = 64
CHUNK = 128
SUB = 2
STAGE = CHUNK * SUB


def _gather_body(table_hbm, idx_hbm, out_hbm,
                 idx_v, rows_a, rows_b, sem_a, sem_b, sem_sa, sem_sb):
    wid = lax.axis_index("s") * NC + lax.axis_index("c")
    chunks_per_w = idx_v.shape[0]
    n_per_w = chunks_per_w * CHUNK
    n_stages = chunks_per_w // SUB
    base = wid * n_per_w

    pltpu.sync_copy(idx_hbm.at[pl.ds(wid * chunks_per_w, chunks_per_w)], idx_v)

    def fire(stage, rows_v, sem):
        for j in range(SUB):
            pltpu.async_copy(
                table_hbm.at[idx_v.at[stage * SUB + j]],
                rows_v.at[pl.ds(j * CHUNK, CHUNK)],
                sem,
            )

    def wait_gathers(stage, rows_v, sem):
        for j in range(SUB):
            pltpu.make_async_copy(
                table_hbm.at[idx_v.at[stage * SUB + j]],
                rows_v.at[pl.ds(j * CHUNK, CHUNK)],
                sem,
            ).wait()

    def store(stage, rows_v, sem):
        pltpu.async_copy(
            rows_v, out_hbm.at[pl.ds(base + stage * STAGE, STAGE)], sem
        )

    def wait_store(stage, rows_v, sem):
        pltpu.make_async_copy(
            rows_v, out_hbm.at[pl.ds(base + stage * STAGE, STAGE)], sem
        ).wait()

    fire(0, rows_a, sem_a)
    fire(1, rows_b, sem_b)
    wait_gathers(0, rows_a, sem_a)
    store(0, rows_a, sem_sa)

    def one_stage(s, cur, cur_gsem, cur_ssem, other, other_gsem, other_ssem):
        wait_store(s - 1, other, other_ssem)

        @pl.when(s + 1 < n_stages)
        def _():
            fire(s + 1, other, other_gsem)

        wait_gathers(s, cur, cur_gsem)
        store(s, cur, cur_ssem)

    def pair(k, carry):
        s = 2 * k + 1
        one_stage(s, rows_b, sem_b, sem_sb, rows_a, sem_a, sem_sa)
        one_stage(s + 1, rows_a, sem_a, sem_sa, rows_b, sem_b, sem_sb)
        return carry

    lax.fori_loop(0, (n_stages - 2) // 2, pair, 0, unroll=False)
    s_last = n_stages - 1
    one_stage(s_last, rows_b, sem_b, sem_sb, rows_a, sem_a, sem_sa)
    wait_store(s_last, rows_b, sem_sb)


def _sc_gather(table2, idx2d):
    n_chunks = idx2d.shape[0]
    n = n_chunks * CHUNK
    chunks_per_w = n_chunks // NW
    mesh = plsc.VectorSubcoreMesh(
        core_axis_name="c", subcore_axis_name="s", num_cores=NC, num_subcores=NS
    )
    return pl.kernel(
        _gather_body,
        out_type=jax.ShapeDtypeStruct((n, DP), jnp.float32),
        mesh=mesh,
        scratch_types=[
            pltpu.VMEM((chunks_per_w, CHUNK), jnp.int32),
            pltpu.VMEM((STAGE, DP), jnp.float32),
            pltpu.VMEM((STAGE, DP), jnp.float32),
            pltpu.SemaphoreType.DMA,
            pltpu.SemaphoreType.DMA,
            pltpu.SemaphoreType.DMA,
            pltpu.SemaphoreType.DMA,
        ],
        name="sc_embedding_gather",
    )(table2, idx2d)


def _dense_body(x_ref, p_ref, w_ref, b_ref, o_ref):
    bm = x_ref.shape[0]
    lane = lax.broadcasted_iota(jnp.int32, (bm, DP), 1)
    keep = (lane >= D) == (p_ref[...] > 0)
    xm = jnp.where(keep, x_ref[...], 0.0)
    acc = jnp.dot(xm, w_ref[...], preferred_element_type=jnp.float32)
    y = jnp.tanh(acc + b_ref[...])
    o_ref[...] = y.reshape(o_ref.shape)


def _dense(x2, par, W2, b, B, L):
    n = x2.shape[0]
    bb = 32
    bm = bb * L
    return pl.pallas_call(
        _dense_body,
        grid=(n // bm,),
        in_specs=[
            pl.BlockSpec((bm, DP), lambda i: (i, 0)),
            pl.BlockSpec((bm, 1), lambda i: (i, 0)),
            pl.BlockSpec((DP, D), lambda i: (0, 0)),
            pl.BlockSpec((1, D), lambda i: (0, 0)),
        ],
        out_specs=pl.BlockSpec((bb, L, D), lambda i: (i, 0, 0)),
        out_shape=jax.ShapeDtypeStruct((B, L, D), jnp.float32),
        name="dense_tanh",
    )(x2, par, W2, b.reshape(1, D))


def kernel(inputs, table, W, b):
    B, L = inputs.shape
    idx = inputs.reshape(-1).astype(jnp.int32)
    table2 = table.reshape(-1, DP)
    idx2d = (idx >> 1).reshape(-1, CHUNK)
    par = (idx & 1).reshape(-1, 1)
    gathered = _sc_gather(table2, idx2d)
    W2 = jnp.concatenate([W, W], axis=0)
    return _dense(gathered, par, W2, b, B, L)


# speedup vs baseline: 1.9608x; 1.9608x over previous
import jax
import jax.numpy as jnp
from jax import lax
from jax.experimental import pallas as pl
from jax.experimental.pallas import tpu as pltpu
from jax.experimental.pallas import tpu_sc as plsc

NC = 2
NS = 16
NW = NC * NS
D = 64
CHUNK = 128
SUB = 4
STAGE = CHUNK * SUB


TK = 8192
HF = TK // 2


def _pack_body(x_ref, y_ref):
    x = x_ref[...]
    y_ref[...] = jnp.concatenate([x[:, :HF].T, x[:, HF:].T], axis=1)


def _pack_table(tableT):
    _, v = tableT.shape
    nblk = pl.cdiv(v, TK)
    return pl.pallas_call(
        _pack_body,
        grid=(nblk,),
        in_specs=[pl.BlockSpec((D, TK), lambda i: (0, i))],
        out_specs=pl.BlockSpec((HF, 2 * D), lambda i: (i, 0)),
        out_shape=jax.ShapeDtypeStruct((nblk * HF, 2 * D), jnp.float32),
        name="table_pack_rowmajor",
    )(tableT)


def _gather_body(table_hbm, idx_hbm, out_hbm,
                 idx_v, rows_a, rows_b, sem_a, sem_b, sem_sa, sem_sb):
    wid = lax.axis_index("s") * NC + lax.axis_index("c")
    chunks_per_w = idx_v.shape[0]
    n_per_w = chunks_per_w * CHUNK
    n_stages = chunks_per_w // SUB
    base = wid * n_per_w

    pltpu.sync_copy(idx_hbm.at[pl.ds(wid * chunks_per_w, chunks_per_w)], idx_v)

    def fire(stage, rows_v, sem):
        for j in range(SUB):
            pltpu.async_copy(
                table_hbm.at[idx_v.at[stage * SUB + j]],
                rows_v.at[pl.ds(j * CHUNK, CHUNK)],
                sem,
            )

    def wait_gathers(stage, rows_v, sem):
        for j in range(SUB):
            pltpu.make_async_copy(
                table_hbm.at[idx_v.at[stage * SUB + j]],
                rows_v.at[pl.ds(j * CHUNK, CHUNK)],
                sem,
            ).wait()

    def store(stage, rows_v, sem):
        pltpu.async_copy(
            rows_v, out_hbm.at[pl.ds(base + stage * STAGE, STAGE)], sem
        )

    def wait_store(stage, rows_v, sem):
        pltpu.make_async_copy(
            rows_v, out_hbm.at[pl.ds(base + stage * STAGE, STAGE)], sem
        ).wait()

    fire(0, rows_a, sem_a)
    fire(1, rows_b, sem_b)
    wait_gathers(0, rows_a, sem_a)
    store(0, rows_a, sem_sa)

    def one_stage(s, cur, cur_gsem, cur_ssem, other, other_gsem, other_ssem):
        wait_store(s - 1, other, other_ssem)

        @pl.when(s + 1 < n_stages)
        def _():
            fire(s + 1, other, other_gsem)

        wait_gathers(s, cur, cur_gsem)
        store(s, cur, cur_ssem)

    def pair(k, carry):
        s = 2 * k + 1
        one_stage(s, rows_b, sem_b, sem_sb, rows_a, sem_a, sem_sa)
        one_stage(s + 1, rows_a, sem_a, sem_sa, rows_b, sem_b, sem_sb)
        return carry

    lax.fori_loop(0, (n_stages - 2) // 2, pair, 0, unroll=False)
    s_last = n_stages - 1
    one_stage(s_last, rows_b, sem_b, sem_sb, rows_a, sem_a, sem_sa)
    wait_store(s_last, rows_b, sem_sb)


def _sc_gather(yv, idx2d):
    n_chunks = idx2d.shape[0]
    n = n_chunks * CHUNK
    chunks_per_w = n_chunks // NW
    mesh = plsc.VectorSubcoreMesh(
        core_axis_name="c", subcore_axis_name="s", num_cores=NC, num_subcores=NS
    )
    return pl.kernel(
        _gather_body,
        out_type=jax.ShapeDtypeStruct((n, D), jnp.float32),
        mesh=mesh,
        scratch_types=[
            pltpu.VMEM((chunks_per_w, CHUNK), jnp.int32),
            pltpu.VMEM((STAGE, D), jnp.float32),
            pltpu.VMEM((STAGE, D), jnp.float32),
            pltpu.SemaphoreType.DMA,
            pltpu.SemaphoreType.DMA,
            pltpu.SemaphoreType.DMA,
            pltpu.SemaphoreType.DMA,
        ],
        compiler_params=pltpu.CompilerParams(use_tc_tiling_on_sc=False),
        name="sc_embedding_gather",
    )(yv, idx2d)


def _project_body(x_ref, w_ref, b_ref, o_ref):
    z = jnp.dot(x_ref[...], w_ref[...], preferred_element_type=jnp.float32)
    y2 = jnp.tanh(z + b_ref[...])
    o_ref[0] = y2[:, :D].T
    o_ref[1] = y2[:, D:].T


def _project(g2, Wd, b2, B, L):
    return pl.pallas_call(
        _project_body,
        grid=(L // 2,),
        in_specs=[
            pl.BlockSpec((B, 2 * D), lambda i: (i, 0)),
            pl.BlockSpec((2 * D, 2 * D), lambda i: (0, 0)),
            pl.BlockSpec((1, 2 * D), lambda i: (0, 0)),
        ],
        out_specs=pl.BlockSpec((2, D, B), lambda i: (i, 0, 0)),
        out_shape=jax.ShapeDtypeStruct((L, D, B), jnp.float32),
        name="project_tanh_to_lanes",
    )(g2, Wd, b2)


def kernel(inputs, table, W, b):
    B, L = inputs.shape
    tableT = table.T
    y2 = _pack_table(tableT)
    yv = y2.reshape(-1, D)

    idxp = inputs.T.astype(jnp.int32)
    idx_pair = jnp.transpose(
        idxp.reshape(L // 2, 2, B), (0, 2, 1)).reshape(-1)
    q = ((idx_pair >> 13) << 12) + (idx_pair & (HF - 1))
    idx_r = 2 * q + ((idx_pair >> 12) & 1)
    idx2d = idx_r.reshape(-1, CHUNK)

    g = _sc_gather(yv, idx2d)
    g2 = g.reshape(B * L // 2, 2 * D)

    Wd = jnp.zeros((2 * D, 2 * D), jnp.float32)
    Wd = Wd.at[:D, :D].set(W).at[D:, D:].set(W)
    b2 = jnp.concatenate([b, b]).reshape(1, 2 * D)

    outT = _project(g2, Wd, b2, B, L)
    return jnp.transpose(outT, (2, 0, 1))


# speedup vs baseline: 2.2894x; 1.1676x over previous
import jax
import jax.numpy as jnp
from jax import lax
from jax.experimental import pallas as pl
from jax.experimental.pallas import tpu as pltpu
from jax.experimental.pallas import tpu_sc as plsc

NC = 2
NS = 16
NW = NC * NS
D = 64
CHUNK = 128
SUB = 4
STAGE = CHUNK * SUB


TK = 8192
HF = TK // 2


def _pack_body(x_ref, i_ref, y_ref):
    x = x_ref[...]
    x2 = jnp.concatenate([x[:, :HF], x[:, HF:]], axis=0)
    y_ref[...] = lax.dot_general(
        x2, i_ref[...], (((0,), (0,)), ((), ())),
        preferred_element_type=jnp.float32)


def _pack_table(tableT, ident):
    _, v = tableT.shape
    nblk = pl.cdiv(v, TK)
    return pl.pallas_call(
        _pack_body,
        grid=(nblk,),
        in_specs=[
            pl.BlockSpec((D, TK), lambda i: (0, i)),
            pl.BlockSpec((2 * D, 2 * D), lambda i: (0, 0)),
        ],
        out_specs=pl.BlockSpec((HF, 2 * D), lambda i: (i, 0)),
        out_shape=jax.ShapeDtypeStruct((nblk * HF, 2 * D), jnp.float32),
        compiler_params=pltpu.CompilerParams(fuse_transposed_lhs_in_matmul=True),
        name="table_pack_rowmajor",
    )(tableT, ident)


def _gather_body(table_hbm, idx_hbm, out_hbm,
                 idx_v, rows_a, rows_b, sem_a, sem_b, sem_sa, sem_sb):
    wid = lax.axis_index("s") * NC + lax.axis_index("c")
    chunks_per_w = idx_v.shape[0]
    n_per_w = chunks_per_w * CHUNK
    n_stages = chunks_per_w // SUB
    base = wid * n_per_w

    pltpu.sync_copy(idx_hbm.at[pl.ds(wid * chunks_per_w, chunks_per_w)], idx_v)

    def fire(stage, rows_v, sem):
        for j in range(SUB):
            pltpu.async_copy(
                table_hbm.at[idx_v.at[stage * SUB + j]],
                rows_v.at[pl.ds(j * CHUNK, CHUNK)],
                sem,
            )

    def wait_gathers(stage, rows_v, sem):
        for j in range(SUB):
            pltpu.make_async_copy(
                table_hbm.at[idx_v.at[stage * SUB + j]],
                rows_v.at[pl.ds(j * CHUNK, CHUNK)],
                sem,
            ).wait()

    def store(stage, rows_v, sem):
        pltpu.async_copy(
            rows_v, out_hbm.at[pl.ds(base + stage * STAGE, STAGE)], sem
        )

    def wait_store(stage, rows_v, sem):
        pltpu.make_async_copy(
            rows_v, out_hbm.at[pl.ds(base + stage * STAGE, STAGE)], sem
        ).wait()

    fire(0, rows_a, sem_a)
    fire(1, rows_b, sem_b)
    wait_gathers(0, rows_a, sem_a)
    store(0, rows_a, sem_sa)

    def one_stage(s, cur, cur_gsem, cur_ssem, other, other_gsem, other_ssem):
        wait_store(s - 1, other, other_ssem)

        @pl.when(s + 1 < n_stages)
        def _():
            fire(s + 1, other, other_gsem)

        wait_gathers(s, cur, cur_gsem)
        store(s, cur, cur_ssem)

    def pair(k, carry):
        s = 2 * k + 1
        one_stage(s, rows_b, sem_b, sem_sb, rows_a, sem_a, sem_sa)
        one_stage(s + 1, rows_a, sem_a, sem_sa, rows_b, sem_b, sem_sb)
        return carry

    lax.fori_loop(0, (n_stages - 2) // 2, pair, 0, unroll=False)
    s_last = n_stages - 1
    one_stage(s_last, rows_b, sem_b, sem_sb, rows_a, sem_a, sem_sa)
    wait_store(s_last, rows_b, sem_sb)


def _sc_gather(yv, idx2d):
    n_chunks = idx2d.shape[0]
    n = n_chunks * CHUNK
    chunks_per_w = n_chunks // NW
    mesh = plsc.VectorSubcoreMesh(
        core_axis_name="c", subcore_axis_name="s", num_cores=NC, num_subcores=NS
    )
    return pl.kernel(
        _gather_body,
        out_type=jax.ShapeDtypeStruct((n, D), jnp.float32),
        mesh=mesh,
        scratch_types=[
            pltpu.VMEM((chunks_per_w, CHUNK), jnp.int32),
            pltpu.VMEM((STAGE, D), jnp.float32),
            pltpu.VMEM((STAGE, D), jnp.float32),
            pltpu.SemaphoreType.DMA,
            pltpu.SemaphoreType.DMA,
            pltpu.SemaphoreType.DMA,
            pltpu.SemaphoreType.DMA,
        ],
        compiler_params=pltpu.CompilerParams(use_tc_tiling_on_sc=False),
        name="sc_embedding_gather",
    )(yv, idx2d)


def _project_body(x_ref, w_ref, b_ref, o_ref):
    z2t = lax.dot_general(
        w_ref[...], x_ref[...], (((0,), (1,)), ((), ())),
        preferred_element_type=jnp.float32)
    y = jnp.tanh(z2t + b_ref[...])
    o_ref[0] = y[:D]
    o_ref[1] = y[D:]


def _project(g2, Wd, b2, B, L):
    return pl.pallas_call(
        _project_body,
        grid=(L // 2,),
        in_specs=[
            pl.BlockSpec((B, 2 * D), lambda i: (i, 0)),
            pl.BlockSpec((2 * D, 2 * D), lambda i: (0, 0)),
            pl.BlockSpec((2 * D, 1), lambda i: (0, 0)),
        ],
        out_specs=pl.BlockSpec((2, D, B), lambda i: (i, 0, 0)),
        out_shape=jax.ShapeDtypeStruct((L, D, B), jnp.float32),
        name="project_tanh_to_lanes",
    )(g2, Wd, b2)


def kernel(inputs, table, W, b):
    B, L = inputs.shape
    tableT = table.T
    ident = jnp.eye(2 * D, dtype=jnp.float32)
    y2 = _pack_table(tableT, ident)
    yv = y2.reshape(-1, D)

    idxp = inputs.T.astype(jnp.int32)
    idx_pair = jnp.transpose(
        idxp.reshape(L // 2, 2, B), (0, 2, 1)).reshape(-1)
    q = ((idx_pair >> 13) << 12) + (idx_pair & (HF - 1))
    idx_r = 2 * q + ((idx_pair >> 12) & 1)
    idx2d = idx_r.reshape(-1, CHUNK)

    g = _sc_gather(yv, idx2d)
    g2 = g.reshape(B * L // 2, 2 * D)

    Wd = jnp.zeros((2 * D, 2 * D), jnp.float32)
    Wd = Wd.at[:D, :D].set(W).at[D:, D:].set(W)
    b2 = jnp.concatenate([b, b]).reshape(2 * D, 1)

    outT = _project(g2, Wd, b2, B, L)
    return jnp.transpose(outT, (2, 0, 1))
